# baseline (speedup 1.0000x reference)
import functools

import jax
import jax.numpy as jnp
from jax import lax
from jax.experimental import pallas as pl
from jax.experimental.pallas import tpu as pltpu
from jax.experimental.pallas import tpu_sc as plsc

_NC = 2
_NS = 16
_NW = _NC * _NS

def _proj_body(n_ref, w_ref, o_ref):
    o_ref[...] = jnp.dot(n_ref[...], w_ref[...],
                         preferred_element_type=jnp.float32)


def _project_tables(nfeat, W1):
    N, D = nfeat.shape
    H = W1.shape[1]
    bn = 1000
    nb = N // bn
    return pl.pallas_call(
        _proj_body,
        grid=(2, nb),
        in_specs=[
            pl.BlockSpec((bn, D), lambda t, n: (n, 0)),
            pl.BlockSpec((D, H), lambda t, n: (t + 1, 0)),
        ],
        out_specs=pl.BlockSpec((bn, H), lambda t, n: (t * nb + n, 0)),
        out_shape=jax.ShapeDtypeStruct((2 * N, H), jnp.float32),
    )(nfeat, W1)


def _make_gather_add(E, H, chunk):
    per_w = E // _NW
    n_chunks = per_w // chunk
    n_pairs = n_chunks // 2
    mesh = plsc.VectorSubcoreMesh(core_axis_name="c", subcore_axis_name="s")

    @functools.partial(
        pl.kernel,
        mesh=mesh,
        out_type=jax.ShapeDtypeStruct((E, H), jnp.float32),
        scratch_types=[
            pltpu.VMEM((n_chunks, chunk), jnp.int32),
            pltpu.VMEM((n_chunks, chunk), jnp.int32),
            pltpu.VMEM((chunk, H), jnp.float32),
            pltpu.VMEM((chunk, H), jnp.float32),
            pltpu.SemaphoreType.DMA,
            pltpu.SemaphoreType.DMA,
        ],
    )
    def gather_add(t_hbm, i1_hbm, i2_hbm, g_hbm,
                   i1s, i2s, ra, rb, sga, sgb):
        wid = lax.axis_index("s") * _NC + lax.axis_index("c")
        base = wid * per_w
        pltpu.sync_copy(i1_hbm.at[wid], i1s)
        pltpu.sync_copy(i2_hbm.at[wid], i2s)

        def issue1(k, rx, semx):
            pltpu.async_copy(t_hbm.at[i1s.at[k]], rx, semx)

        def issue2(k, rx, semx):
            pltpu.async_copy(t_hbm.at[i2s.at[k]], rx, semx, add=True)

        def wait(rx, semx):
            pltpu.make_async_copy(t_hbm.at[pl.ds(0, chunk)], rx, semx).wait()

        def store(k, rx):
            pltpu.sync_copy(rx, g_hbm.at[pl.ds(base + k * chunk, chunk)])

        issue1(0, ra, sga)

        def pair(q, c):
            k0 = 2 * q
            issue1(k0 + 1, rb, sgb)
            wait(ra, sga)
            issue2(k0, ra, sga)
            wait(ra, sga)
            store(k0, ra)
            issue1(k0 + 2, ra, sga)
            wait(rb, sgb)
            issue2(k0 + 1, rb, sgb)
            wait(rb, sgb)
            store(k0 + 1, rb)
            return c

        if n_chunks % 2 == 0:
            lax.fori_loop(0, n_pairs - 1, pair, 0)
            k0 = n_chunks - 2
            issue1(k0 + 1, rb, sgb)
            wait(ra, sga)
            issue2(k0, ra, sga)
            wait(ra, sga)
            store(k0, ra)
            wait(rb, sgb)
            issue2(k0 + 1, rb, sgb)
            wait(rb, sgb)
            store(k0 + 1, rb)
        else:
            lax.fori_loop(0, n_pairs - 1, pair, 0)
            k0 = n_chunks - 3
            issue1(k0 + 1, rb, sgb)
            wait(ra, sga)
            issue2(k0, ra, sga)
            wait(ra, sga)
            store(k0, ra)
            issue1(k0 + 2, ra, sga)
            wait(rb, sgb)
            issue2(k0 + 1, rb, sgb)
            wait(rb, sgb)
            store(k0 + 1, rb)
            wait(ra, sga)
            issue2(k0 + 2, ra, sga)
            wait(ra, sga)
            store(k0 + 2, ra)

    return gather_add


def _mlp_body(a_ref, e_ref, g_ref, w1_ref, b1_ref, w2_ref, b2_ref,
              gm_ref, bt_ref, o_ref):
    del a_ref
    x = e_ref[...]
    z = (jnp.dot(x.astype(jnp.bfloat16), w1_ref[...],
                 preferred_element_type=jnp.float32)
         + g_ref[...] + b1_ref[...])
    h1 = z * jax.nn.sigmoid(z)
    h2 = (jnp.dot(h1.astype(jnp.bfloat16), w2_ref[...],
                  preferred_element_type=jnp.float32)
          + b2_ref[...])
    mu = jnp.mean(h2, axis=-1, keepdims=True)
    d = h2 - mu
    var = jnp.mean(d * d, axis=-1, keepdims=True)
    y = d * lax.rsqrt(var + 1e-5) * gm_ref[...] + bt_ref[...] + x
    o_ref[...] = y


def _edge_mlp_chunk(acc, efeat, g, W1, b1, W2, b2, gamma, beta, c, be):
    E, D = efeat.shape
    Ec, H = g.shape
    nb = Ec // be
    off = c * nb
    body = _mlp_body if acc is not None else functools.partial(_mlp_body, None)
    specs = [
        pl.BlockSpec((be, D), lambda i: (off + i, 0)),
        pl.BlockSpec((be, H), lambda i: (i, 0)),
        pl.BlockSpec((D, H), lambda i: (0, 0)),
        pl.BlockSpec((1, H), lambda i: (0, 0)),
        pl.BlockSpec((H, D), lambda i: (0, 0)),
        pl.BlockSpec((1, D), lambda i: (0, 0)),
        pl.BlockSpec((1, D), lambda i: (0, 0)),
        pl.BlockSpec((1, D), lambda i: (0, 0)),
    ]
    args = (efeat, g, W1, b1, W2, b2, gamma, beta)
    if acc is not None:
        specs = [pl.BlockSpec(memory_space=pl.ANY)] + specs
        args = (acc,) + args
    return pl.pallas_call(
        body,
        grid=(nb,),
        in_specs=specs,
        out_specs=pl.BlockSpec((be, D), lambda i: (off + i, 0)),
        out_shape=jax.ShapeDtypeStruct((E, D), jnp.float32),
        input_output_aliases={0: 0} if acc is not None else {},
    )(*args)


def kernel(efeat, nfeat, edge_index, W1, b1, W2, b2, gamma, beta):
    E, D = efeat.shape
    N = nfeat.shape[0]
    H = W1.shape[1]
    K = 5
    CH = 80
    BE = 1600
    Ec = E // K
    per_w = Ec // _NW
    n_chunks = per_w // CH

    T = _project_tables(nfeat, W1)
    gather = _make_gather_add(Ec, H, CH)
    w1e = W1[:D].astype(jnp.bfloat16)
    w2 = W2.astype(jnp.bfloat16)
    b1r = b1.reshape(1, H)
    b2r = b2.reshape(1, D)
    gm = gamma.reshape(1, D)
    bt = beta.reshape(1, D)
    acc = None
    for c in range(K):
        sl = slice(c * Ec, (c + 1) * Ec)
        isrc = edge_index[0, sl].reshape(_NW, n_chunks, CH)
        idst = (edge_index[1, sl] + N).reshape(_NW, n_chunks, CH)
        g = gather(T, isrc, idst)
        acc = _edge_mlp_chunk(acc, efeat, g, w1e, b1r, w2, b2r, gm, bt,
                              c, BE)
    return acc


# speedup vs baseline: 1.0201x; 1.0201x over previous
import functools

import jax
import jax.numpy as jnp
from jax import lax
from jax.experimental import pallas as pl
from jax.experimental.pallas import tpu as pltpu
from jax.experimental.pallas import tpu_sc as plsc

_NC = 2
_NS = 16
_NW = _NC * _NS

def _proj_body(n_ref, w_ref, o_ref):
    o_ref[...] = jnp.dot(n_ref[...], w_ref[...],
                         preferred_element_type=jnp.float32)


def _project_tables(nfeat, W1):
    N, D = nfeat.shape
    H = W1.shape[1]
    bn = 1000
    nb = N // bn
    return pl.pallas_call(
        _proj_body,
        grid=(2, nb),
        in_specs=[
            pl.BlockSpec((bn, D), lambda t, n: (n, 0)),
            pl.BlockSpec((D, H), lambda t, n: (t + 1, 0)),
        ],
        out_specs=pl.BlockSpec((bn, H), lambda t, n: (t * nb + n, 0)),
        out_shape=jax.ShapeDtypeStruct((2 * N, H), jnp.float32),
    )(nfeat, W1)


def _make_gather_add(E, H, chunk):
    per_w = E // _NW
    n_chunks = per_w // chunk
    n_pairs = n_chunks // 2
    mesh = plsc.VectorSubcoreMesh(core_axis_name="c", subcore_axis_name="s")

    @functools.partial(
        pl.kernel,
        mesh=mesh,
        out_type=jax.ShapeDtypeStruct((E, H), jnp.float32),
        scratch_types=[
            pltpu.VMEM((n_chunks, chunk), jnp.int32),
            pltpu.VMEM((n_chunks, chunk), jnp.int32),
            pltpu.VMEM((chunk, H), jnp.float32),
            pltpu.VMEM((chunk, H), jnp.float32),
            pltpu.SemaphoreType.DMA,
            pltpu.SemaphoreType.DMA,
        ],
    )
    def gather_add(t_hbm, i1_hbm, i2_hbm, g_hbm,
                   i1s, i2s, ra, rb, sga, sgb):
        wid = lax.axis_index("s") * _NC + lax.axis_index("c")
        base = wid * per_w
        pltpu.sync_copy(i1_hbm.at[wid], i1s)
        pltpu.sync_copy(i2_hbm.at[wid], i2s)

        def issue1(k, rx, semx):
            pltpu.async_copy(t_hbm.at[i1s.at[k]], rx, semx)

        def issue2(k, rx, semx):
            pltpu.async_copy(t_hbm.at[i2s.at[k]], rx, semx, add=True)

        def wait(rx, semx):
            pltpu.make_async_copy(t_hbm.at[pl.ds(0, chunk)], rx, semx).wait()

        def store(k, rx):
            pltpu.sync_copy(rx, g_hbm.at[pl.ds(base + k * chunk, chunk)])

        issue1(0, ra, sga)

        def pair(q, c):
            k0 = 2 * q
            issue1(k0 + 1, rb, sgb)
            wait(ra, sga)
            issue2(k0, ra, sga)
            wait(ra, sga)
            store(k0, ra)
            issue1(k0 + 2, ra, sga)
            wait(rb, sgb)
            issue2(k0 + 1, rb, sgb)
            wait(rb, sgb)
            store(k0 + 1, rb)
            return c

        if n_chunks % 2 == 0:
            lax.fori_loop(0, n_pairs - 1, pair, 0)
            k0 = n_chunks - 2
            issue1(k0 + 1, rb, sgb)
            wait(ra, sga)
            issue2(k0, ra, sga)
            wait(ra, sga)
            store(k0, ra)
            wait(rb, sgb)
            issue2(k0 + 1, rb, sgb)
            wait(rb, sgb)
            store(k0 + 1, rb)
        else:
            lax.fori_loop(0, n_pairs - 1, pair, 0)
            k0 = n_chunks - 3
            issue1(k0 + 1, rb, sgb)
            wait(ra, sga)
            issue2(k0, ra, sga)
            wait(ra, sga)
            store(k0, ra)
            issue1(k0 + 2, ra, sga)
            wait(rb, sgb)
            issue2(k0 + 1, rb, sgb)
            wait(rb, sgb)
            store(k0 + 1, rb)
            wait(ra, sga)
            issue2(k0 + 2, ra, sga)
            wait(ra, sga)
            store(k0 + 2, ra)

    return gather_add


def _mlp_body(a_ref, e_ref, g_ref, w1_ref, b1_ref, w2_ref, b2_ref,
              gm_ref, bt_ref, o_ref):
    del a_ref
    x = e_ref[...]
    z = (jnp.dot(x.astype(jnp.bfloat16), w1_ref[...],
                 preferred_element_type=jnp.float32)
         + g_ref[...] + b1_ref[...])
    h1 = z * jax.nn.sigmoid(z)
    h2 = (jnp.dot(h1.astype(jnp.bfloat16), w2_ref[...],
                  preferred_element_type=jnp.float32)
          + b2_ref[...])
    mu = jnp.mean(h2, axis=-1, keepdims=True)
    d = h2 - mu
    var = jnp.mean(d * d, axis=-1, keepdims=True)
    y = d * lax.rsqrt(var + 1e-5) * gm_ref[...] + bt_ref[...] + x
    o_ref[...] = y


def _edge_mlp_chunk(acc, efeat, g, W1, b1, W2, b2, gamma, beta, c, be):
    E, D = efeat.shape
    Ec, H = g.shape
    nb = Ec // be
    off = c * nb
    body = _mlp_body if acc is not None else functools.partial(_mlp_body, None)
    specs = [
        pl.BlockSpec((be, D), lambda i: (off + i, 0)),
        pl.BlockSpec((be, H), lambda i: (i, 0)),
        pl.BlockSpec((D, H), lambda i: (0, 0)),
        pl.BlockSpec((1, H), lambda i: (0, 0)),
        pl.BlockSpec((H, D), lambda i: (0, 0)),
        pl.BlockSpec((1, D), lambda i: (0, 0)),
        pl.BlockSpec((1, D), lambda i: (0, 0)),
        pl.BlockSpec((1, D), lambda i: (0, 0)),
    ]
    args = (efeat, g, W1, b1, W2, b2, gamma, beta)
    if acc is not None:
        specs = [pl.BlockSpec(memory_space=pl.ANY)] + specs
        args = (acc,) + args
    return pl.pallas_call(
        body,
        grid=(nb,),
        in_specs=specs,
        out_specs=pl.BlockSpec((be, D), lambda i: (off + i, 0)),
        out_shape=jax.ShapeDtypeStruct((E, D), jnp.float32),
        input_output_aliases={0: 0} if acc is not None else {},
    )(*args)


def kernel(efeat, nfeat, edge_index, W1, b1, W2, b2, gamma, beta):
    E, D = efeat.shape
    N = nfeat.shape[0]
    H = W1.shape[1]
    K = 5
    CH = 80
    BE = 3200
    Ec = E // K
    per_w = Ec // _NW
    n_chunks = per_w // CH

    T = _project_tables(nfeat, W1)
    gather = _make_gather_add(Ec, H, CH)
    w1e = W1[:D].astype(jnp.bfloat16)
    w2 = W2.astype(jnp.bfloat16)
    b1r = b1.reshape(1, H)
    b2r = b2.reshape(1, D)
    gm = gamma.reshape(1, D)
    bt = beta.reshape(1, D)
    acc = None
    for c in range(K):
        sl = slice(c * Ec, (c + 1) * Ec)
        isrc = edge_index[0, sl].reshape(_NW, n_chunks, CH)
        idst = (edge_index[1, sl] + N).reshape(_NW, n_chunks, CH)
        g = gather(T, isrc, idst)
        acc = _edge_mlp_chunk(acc, efeat, g, w1e, b1r, w2, b2r, gm, bt,
                              c, BE)
    return acc


# speedup vs baseline: 1.0272x; 1.0070x over previous
import functools

import jax
import jax.numpy as jnp
from jax import lax
from jax.experimental import pallas as pl
from jax.experimental.pallas import tpu as pltpu
from jax.experimental.pallas import tpu_sc as plsc

_NC = 2
_NS = 16
_NW = _NC * _NS

def _proj_body(n_ref, w_ref, o_ref):
    o_ref[...] = jnp.dot(n_ref[...], w_ref[...],
                         preferred_element_type=jnp.float32)


def _project_tables(nfeat, W1):
    N, D = nfeat.shape
    H = W1.shape[1]
    bn = 1000
    nb = N // bn
    return pl.pallas_call(
        _proj_body,
        grid=(2, nb),
        in_specs=[
            pl.BlockSpec((bn, D), lambda t, n: (n, 0)),
            pl.BlockSpec((D, H), lambda t, n: (t + 1, 0)),
        ],
        out_specs=pl.BlockSpec((bn, H), lambda t, n: (t * nb + n, 0)),
        out_shape=jax.ShapeDtypeStruct((2 * N, H), jnp.float32),
    )(nfeat, W1)


def _make_gather_add(E, H, chunk):
    per_w = E // _NW
    n_chunks = per_w // chunk
    n_pairs = n_chunks // 2
    mesh = plsc.VectorSubcoreMesh(core_axis_name="c", subcore_axis_name="s")

    @functools.partial(
        pl.kernel,
        mesh=mesh,
        out_type=jax.ShapeDtypeStruct((E, H), jnp.float32),
        scratch_types=[
            pltpu.VMEM((n_chunks, chunk), jnp.int32),
            pltpu.VMEM((n_chunks, chunk), jnp.int32),
            pltpu.VMEM((chunk, H), jnp.float32),
            pltpu.VMEM((chunk, H), jnp.float32),
            pltpu.SemaphoreType.DMA,
            pltpu.SemaphoreType.DMA,
        ],
    )
    def gather_add(t_hbm, i1_hbm, i2_hbm, g_hbm,
                   i1s, i2s, ra, rb, sga, sgb):
        wid = lax.axis_index("s") * _NC + lax.axis_index("c")
        base = wid * per_w
        pltpu.sync_copy(i1_hbm.at[wid], i1s)
        pltpu.sync_copy(i2_hbm.at[wid], i2s)

        def issue1(k, rx, semx):
            pltpu.async_copy(t_hbm.at[i1s.at[k]], rx, semx)

        def issue2(k, rx, semx):
            pltpu.async_copy(t_hbm.at[i2s.at[k]], rx, semx, add=True)

        def wait(rx, semx):
            pltpu.make_async_copy(t_hbm.at[pl.ds(0, chunk)], rx, semx).wait()

        def store(k, rx):
            pltpu.sync_copy(rx, g_hbm.at[pl.ds(base + k * chunk, chunk)])

        issue1(0, ra, sga)

        def pair(q, c):
            k0 = 2 * q
            issue1(k0 + 1, rb, sgb)
            wait(ra, sga)
            issue2(k0, ra, sga)
            wait(ra, sga)
            store(k0, ra)
            issue1(k0 + 2, ra, sga)
            wait(rb, sgb)
            issue2(k0 + 1, rb, sgb)
            wait(rb, sgb)
            store(k0 + 1, rb)
            return c

        if n_chunks % 2 == 0:
            lax.fori_loop(0, n_pairs - 1, pair, 0)
            k0 = n_chunks - 2
            issue1(k0 + 1, rb, sgb)
            wait(ra, sga)
            issue2(k0, ra, sga)
            wait(ra, sga)
            store(k0, ra)
            wait(rb, sgb)
            issue2(k0 + 1, rb, sgb)
            wait(rb, sgb)
            store(k0 + 1, rb)
        else:
            lax.fori_loop(0, n_pairs - 1, pair, 0)
            k0 = n_chunks - 3
            issue1(k0 + 1, rb, sgb)
            wait(ra, sga)
            issue2(k0, ra, sga)
            wait(ra, sga)
            store(k0, ra)
            issue1(k0 + 2, ra, sga)
            wait(rb, sgb)
            issue2(k0 + 1, rb, sgb)
            wait(rb, sgb)
            store(k0 + 1, rb)
            wait(ra, sga)
            issue2(k0 + 2, ra, sga)
            wait(ra, sga)
            store(k0 + 2, ra)

    return gather_add


def _mlp_body(a_ref, e_ref, g_ref, w1_ref, b1_ref, w2_ref, b2_ref,
              gm_ref, bt_ref, o_ref):
    del a_ref
    x = e_ref[...]
    z = (jnp.dot(x.astype(jnp.bfloat16), w1_ref[...],
                 preferred_element_type=jnp.float32)
         + g_ref[...] + b1_ref[...])
    h1 = z * jax.nn.sigmoid(z)
    h2 = (jnp.dot(h1.astype(jnp.bfloat16), w2_ref[...],
                  preferred_element_type=jnp.float32)
          + b2_ref[...])
    mu = jnp.mean(h2, axis=-1, keepdims=True)
    d = h2 - mu
    var = jnp.mean(d * d, axis=-1, keepdims=True)
    y = d * lax.rsqrt(var + 1e-5) * gm_ref[...] + bt_ref[...] + x
    o_ref[...] = y


def _edge_mlp_chunk(acc, efeat, g, W1, b1, W2, b2, gamma, beta, c, be):
    E, D = efeat.shape
    Ec, H = g.shape
    nb = Ec // be
    off = c * nb
    body = _mlp_body if acc is not None else functools.partial(_mlp_body, None)
    specs = [
        pl.BlockSpec((be, D), lambda i: (off + i, 0)),
        pl.BlockSpec((be, H), lambda i: (i, 0)),
        pl.BlockSpec((D, H), lambda i: (0, 0)),
        pl.BlockSpec((1, H), lambda i: (0, 0)),
        pl.BlockSpec((H, D), lambda i: (0, 0)),
        pl.BlockSpec((1, D), lambda i: (0, 0)),
        pl.BlockSpec((1, D), lambda i: (0, 0)),
        pl.BlockSpec((1, D), lambda i: (0, 0)),
    ]
    args = (efeat, g, W1, b1, W2, b2, gamma, beta)
    if acc is not None:
        specs = [pl.BlockSpec(memory_space=pl.ANY)] + specs
        args = (acc,) + args
    return pl.pallas_call(
        body,
        grid=(nb,),
        in_specs=specs,
        out_specs=pl.BlockSpec((be, D), lambda i: (off + i, 0)),
        out_shape=jax.ShapeDtypeStruct((E, D), jnp.float32),
        input_output_aliases={0: 0} if acc is not None else {},
    )(*args)


def kernel(efeat, nfeat, edge_index, W1, b1, W2, b2, gamma, beta):
    E, D = efeat.shape
    N = nfeat.shape[0]
    H = W1.shape[1]
    K = 5
    CH = 80
    BE = 6400
    Ec = E // K
    per_w = Ec // _NW
    n_chunks = per_w // CH

    T = _project_tables(nfeat, W1)
    gather = _make_gather_add(Ec, H, CH)
    w1e = W1[:D].astype(jnp.bfloat16)
    w2 = W2.astype(jnp.bfloat16)
    b1r = b1.reshape(1, H)
    b2r = b2.reshape(1, D)
    gm = gamma.reshape(1, D)
    bt = beta.reshape(1, D)
    acc = None
    for c in range(K):
        sl = slice(c * Ec, (c + 1) * Ec)
        isrc = edge_index[0, sl].reshape(_NW, n_chunks, CH)
        idst = (edge_index[1, sl] + N).reshape(_NW, n_chunks, CH)
        g = gather(T, isrc, idst)
        acc = _edge_mlp_chunk(acc, efeat, g, w1e, b1r, w2, b2r, gm, bt,
                              c, BE)
    return acc


# speedup vs baseline: 1.0337x; 1.0062x over previous
import functools

import jax
import jax.numpy as jnp
from jax import lax
from jax.experimental import pallas as pl
from jax.experimental.pallas import tpu as pltpu
from jax.experimental.pallas import tpu_sc as plsc

_NC = 2
_NS = 16
_NW = _NC * _NS

def _proj_body(n_ref, w_ref, o_ref):
    o_ref[...] = jnp.dot(n_ref[...], w_ref[...],
                         preferred_element_type=jnp.float32)


def _project_tables(nfeat, W1):
    N, D = nfeat.shape
    H = W1.shape[1]
    bn = 1000
    nb = N // bn
    return pl.pallas_call(
        _proj_body,
        grid=(2, nb),
        in_specs=[
            pl.BlockSpec((bn, D), lambda t, n: (n, 0)),
            pl.BlockSpec((D, H), lambda t, n: (t + 1, 0)),
        ],
        out_specs=pl.BlockSpec((bn, H), lambda t, n: (t * nb + n, 0)),
        out_shape=jax.ShapeDtypeStruct((2 * N, H), jnp.float32),
    )(nfeat, W1)


def _make_gather_add(E, H, chunk):
    per_w = E // _NW
    n_chunks = per_w // chunk
    n_pairs = n_chunks // 2
    mesh = plsc.VectorSubcoreMesh(core_axis_name="c", subcore_axis_name="s")

    @functools.partial(
        pl.kernel,
        mesh=mesh,
        out_type=jax.ShapeDtypeStruct((E, H), jnp.float32),
        scratch_types=[
            pltpu.VMEM((n_chunks, chunk), jnp.int32),
            pltpu.VMEM((n_chunks, chunk), jnp.int32),
            pltpu.VMEM((chunk, H), jnp.float32),
            pltpu.VMEM((chunk, H), jnp.float32),
            pltpu.SemaphoreType.DMA,
            pltpu.SemaphoreType.DMA,
        ],
    )
    def gather_add(t_hbm, i1_hbm, i2_hbm, g_hbm,
                   i1s, i2s, ra, rb, sga, sgb):
        wid = lax.axis_index("s") * _NC + lax.axis_index("c")
        base = wid * per_w
        pltpu.sync_copy(i1_hbm.at[wid], i1s)
        pltpu.sync_copy(i2_hbm.at[wid], i2s)

        def issue1(k, rx, semx):
            pltpu.async_copy(t_hbm.at[i1s.at[k]], rx, semx)

        def issue2(k, rx, semx):
            pltpu.async_copy(t_hbm.at[i2s.at[k]], rx, semx, add=True)

        def wait(rx, semx):
            pltpu.make_async_copy(t_hbm.at[pl.ds(0, chunk)], rx, semx).wait()

        def store(k, rx):
            pltpu.sync_copy(rx, g_hbm.at[pl.ds(base + k * chunk, chunk)])

        issue1(0, ra, sga)

        def pair(q, c):
            k0 = 2 * q
            issue1(k0 + 1, rb, sgb)
            wait(ra, sga)
            issue2(k0, ra, sga)
            wait(ra, sga)
            store(k0, ra)
            issue1(k0 + 2, ra, sga)
            wait(rb, sgb)
            issue2(k0 + 1, rb, sgb)
            wait(rb, sgb)
            store(k0 + 1, rb)
            return c

        if n_chunks % 2 == 0:
            lax.fori_loop(0, n_pairs - 1, pair, 0)
            k0 = n_chunks - 2
            issue1(k0 + 1, rb, sgb)
            wait(ra, sga)
            issue2(k0, ra, sga)
            wait(ra, sga)
            store(k0, ra)
            wait(rb, sgb)
            issue2(k0 + 1, rb, sgb)
            wait(rb, sgb)
            store(k0 + 1, rb)
        else:
            lax.fori_loop(0, n_pairs - 1, pair, 0)
            k0 = n_chunks - 3
            issue1(k0 + 1, rb, sgb)
            wait(ra, sga)
            issue2(k0, ra, sga)
            wait(ra, sga)
            store(k0, ra)
            issue1(k0 + 2, ra, sga)
            wait(rb, sgb)
            issue2(k0 + 1, rb, sgb)
            wait(rb, sgb)
            store(k0 + 1, rb)
            wait(ra, sga)
            issue2(k0 + 2, ra, sga)
            wait(ra, sga)
            store(k0 + 2, ra)

    return gather_add


def _mlp_body(a_ref, e_ref, g_ref, w1_ref, b1_ref, w2_ref, b2_ref,
              gm_ref, bt_ref, o_ref):
    del a_ref
    x = e_ref[...]
    z = (jnp.dot(x.astype(jnp.bfloat16), w1_ref[...],
                 preferred_element_type=jnp.float32)
         + g_ref[...] + b1_ref[...])
    h1 = z * jax.nn.sigmoid(z)
    h2 = (jnp.dot(h1.astype(jnp.bfloat16), w2_ref[...],
                  preferred_element_type=jnp.float32)
          + b2_ref[...])
    mu = jnp.mean(h2, axis=-1, keepdims=True)
    d = h2 - mu
    var = jnp.mean(d * d, axis=-1, keepdims=True)
    y = d * lax.rsqrt(var + 1e-5) * gm_ref[...] + bt_ref[...] + x
    o_ref[...] = y


def _edge_mlp_chunk(acc, efeat, g, W1, b1, W2, b2, gamma, beta, c, be):
    E, D = efeat.shape
    Ec, H = g.shape
    nb = Ec // be
    off = c * nb
    body = _mlp_body if acc is not None else functools.partial(_mlp_body, None)
    specs = [
        pl.BlockSpec((be, D), lambda i: (off + i, 0)),
        pl.BlockSpec((be, H), lambda i: (i, 0)),
        pl.BlockSpec((D, H), lambda i: (0, 0)),
        pl.BlockSpec((1, H), lambda i: (0, 0)),
        pl.BlockSpec((H, D), lambda i: (0, 0)),
        pl.BlockSpec((1, D), lambda i: (0, 0)),
        pl.BlockSpec((1, D), lambda i: (0, 0)),
        pl.BlockSpec((1, D), lambda i: (0, 0)),
    ]
    args = (efeat, g, W1, b1, W2, b2, gamma, beta)
    if acc is not None:
        specs = [pl.BlockSpec(memory_space=pl.ANY)] + specs
        args = (acc,) + args
    return pl.pallas_call(
        body,
        grid=(nb,),
        in_specs=specs,
        out_specs=pl.BlockSpec((be, D), lambda i: (off + i, 0)),
        out_shape=jax.ShapeDtypeStruct((E, D), jnp.float32),
        input_output_aliases={0: 0} if acc is not None else {},
    )(*args)


def kernel(efeat, nfeat, edge_index, W1, b1, W2, b2, gamma, beta):
    E, D = efeat.shape
    N = nfeat.shape[0]
    H = W1.shape[1]
    K = 5
    CH = 80
    BE = 12800
    Ec = E // K
    per_w = Ec // _NW
    n_chunks = per_w // CH

    T = _project_tables(nfeat, W1)
    gather = _make_gather_add(Ec, H, CH)
    w1e = W1[:D].astype(jnp.bfloat16)
    w2 = W2.astype(jnp.bfloat16)
    b1r = b1.reshape(1, H)
    b2r = b2.reshape(1, D)
    gm = gamma.reshape(1, D)
    bt = beta.reshape(1, D)
    acc = None
    for c in range(K):
        sl = slice(c * Ec, (c + 1) * Ec)
        isrc = edge_index[0, sl].reshape(_NW, n_chunks, CH)
        idst = (edge_index[1, sl] + N).reshape(_NW, n_chunks, CH)
        g = gather(T, isrc, idst)
        acc = _edge_mlp_chunk(acc, efeat, g, w1e, b1r, w2, b2r, gm, bt,
                              c, BE)
    return acc


# speedup vs baseline: 1.0404x; 1.0065x over previous
import functools

import jax
import jax.numpy as jnp
from jax import lax
from jax.experimental import pallas as pl
from jax.experimental.pallas import tpu as pltpu
from jax.experimental.pallas import tpu_sc as plsc

_NC = 2
_NS = 16
_NW = _NC * _NS

def _proj_body(n_ref, w_ref, o_ref):
    o_ref[...] = jnp.dot(n_ref[...], w_ref[...],
                         preferred_element_type=jnp.float32)


def _project_tables(nfeat, W1):
    N, D = nfeat.shape
    H = W1.shape[1]
    bn = 1000
    nb = N // bn
    return pl.pallas_call(
        _proj_body,
        grid=(2, nb),
        in_specs=[
            pl.BlockSpec((bn, D), lambda t, n: (n, 0)),
            pl.BlockSpec((D, H), lambda t, n: (t + 1, 0)),
        ],
        out_specs=pl.BlockSpec((bn, H), lambda t, n: (t * nb + n, 0)),
        out_shape=jax.ShapeDtypeStruct((2 * N, H), jnp.float32),
    )(nfeat, W1)


def _make_gather_add(E, H, chunk):
    per_w = E // _NW
    n_chunks = per_w // chunk
    n_pairs = n_chunks // 2
    mesh = plsc.VectorSubcoreMesh(core_axis_name="c", subcore_axis_name="s")

    @functools.partial(
        pl.kernel,
        mesh=mesh,
        out_type=jax.ShapeDtypeStruct((E, H), jnp.float32),
        scratch_types=[
            pltpu.VMEM((n_chunks, chunk), jnp.int32),
            pltpu.VMEM((n_chunks, chunk), jnp.int32),
            pltpu.VMEM((chunk, H), jnp.float32),
            pltpu.VMEM((chunk, H), jnp.float32),
            pltpu.SemaphoreType.DMA,
            pltpu.SemaphoreType.DMA,
        ],
    )
    def gather_add(t_hbm, i1_hbm, i2_hbm, g_hbm,
                   i1s, i2s, ra, rb, sga, sgb):
        wid = lax.axis_index("s") * _NC + lax.axis_index("c")
        base = wid * per_w
        pltpu.sync_copy(i1_hbm.at[wid], i1s)
        pltpu.sync_copy(i2_hbm.at[wid], i2s)

        def issue1(k, rx, semx):
            pltpu.async_copy(t_hbm.at[i1s.at[k]], rx, semx)

        def issue2(k, rx, semx):
            pltpu.async_copy(t_hbm.at[i2s.at[k]], rx, semx, add=True)

        def wait(rx, semx):
            pltpu.make_async_copy(t_hbm.at[pl.ds(0, chunk)], rx, semx).wait()

        def store(k, rx):
            pltpu.sync_copy(rx, g_hbm.at[pl.ds(base + k * chunk, chunk)])

        issue1(0, ra, sga)

        def pair(q, c):
            k0 = 2 * q
            issue1(k0 + 1, rb, sgb)
            wait(ra, sga)
            issue2(k0, ra, sga)
            wait(ra, sga)
            store(k0, ra)
            issue1(k0 + 2, ra, sga)
            wait(rb, sgb)
            issue2(k0 + 1, rb, sgb)
            wait(rb, sgb)
            store(k0 + 1, rb)
            return c

        if n_chunks % 2 == 0:
            lax.fori_loop(0, n_pairs - 1, pair, 0)
            k0 = n_chunks - 2
            issue1(k0 + 1, rb, sgb)
            wait(ra, sga)
            issue2(k0, ra, sga)
            wait(ra, sga)
            store(k0, ra)
            wait(rb, sgb)
            issue2(k0 + 1, rb, sgb)
            wait(rb, sgb)
            store(k0 + 1, rb)
        else:
            lax.fori_loop(0, n_pairs - 1, pair, 0)
            k0 = n_chunks - 3
            issue1(k0 + 1, rb, sgb)
            wait(ra, sga)
            issue2(k0, ra, sga)
            wait(ra, sga)
            store(k0, ra)
            issue1(k0 + 2, ra, sga)
            wait(rb, sgb)
            issue2(k0 + 1, rb, sgb)
            wait(rb, sgb)
            store(k0 + 1, rb)
            wait(ra, sga)
            issue2(k0 + 2, ra, sga)
            wait(ra, sga)
            store(k0 + 2, ra)

    return gather_add


def _mlp_body(a_ref, e_ref, g_ref, w1_ref, b1_ref, w2_ref, b2_ref,
              gm_ref, bt_ref, o_ref):
    del a_ref
    x = e_ref[...]
    z = (jnp.dot(x.astype(jnp.bfloat16), w1_ref[...],
                 preferred_element_type=jnp.float32)
         + g_ref[...] + b1_ref[...])
    h1 = z * jax.nn.sigmoid(z)
    h2 = (jnp.dot(h1.astype(jnp.bfloat16), w2_ref[...],
                  preferred_element_type=jnp.float32)
          + b2_ref[...])
    mu = jnp.mean(h2, axis=-1, keepdims=True)
    d = h2 - mu
    var = jnp.mean(d * d, axis=-1, keepdims=True)
    y = d * lax.rsqrt(var + 1e-5) * gm_ref[...] + bt_ref[...] + x
    o_ref[...] = y


def _edge_mlp_chunk(acc, efeat, g, W1, b1, W2, b2, gamma, beta, c, be):
    E, D = efeat.shape
    Ec, H = g.shape
    nb = Ec // be
    off = c * nb
    body = _mlp_body if acc is not None else functools.partial(_mlp_body, None)
    specs = [
        pl.BlockSpec((be, D), lambda i: (off + i, 0)),
        pl.BlockSpec((be, H), lambda i: (i, 0)),
        pl.BlockSpec((D, H), lambda i: (0, 0)),
        pl.BlockSpec((1, H), lambda i: (0, 0)),
        pl.BlockSpec((H, D), lambda i: (0, 0)),
        pl.BlockSpec((1, D), lambda i: (0, 0)),
        pl.BlockSpec((1, D), lambda i: (0, 0)),
        pl.BlockSpec((1, D), lambda i: (0, 0)),
    ]
    args = (efeat, g, W1, b1, W2, b2, gamma, beta)
    if acc is not None:
        specs = [pl.BlockSpec(memory_space=pl.ANY)] + specs
        args = (acc,) + args
    return pl.pallas_call(
        body,
        grid=(nb,),
        in_specs=specs,
        out_specs=pl.BlockSpec((be, D), lambda i: (off + i, 0)),
        out_shape=jax.ShapeDtypeStruct((E, D), jnp.float32),
        input_output_aliases={0: 0} if acc is not None else {},
    )(*args)


def kernel(efeat, nfeat, edge_index, W1, b1, W2, b2, gamma, beta):
    E, D = efeat.shape
    N = nfeat.shape[0]
    H = W1.shape[1]
    K = 5
    CH = 80
    BE = 16000
    Ec = E // K
    per_w = Ec // _NW
    n_chunks = per_w // CH

    T = _project_tables(nfeat, W1)
    gather = _make_gather_add(Ec, H, CH)
    w1e = W1[:D].astype(jnp.bfloat16)
    w2 = W2.astype(jnp.bfloat16)
    b1r = b1.reshape(1, H)
    b2r = b2.reshape(1, D)
    gm = gamma.reshape(1, D)
    bt = beta.reshape(1, D)
    acc = None
    for c in range(K):
        sl = slice(c * Ec, (c + 1) * Ec)
        isrc = edge_index[0, sl].reshape(_NW, n_chunks, CH)
        idst = (edge_index[1, sl] + N).reshape(_NW, n_chunks, CH)
        g = gather(T, isrc, idst)
        acc = _edge_mlp_chunk(acc, efeat, g, w1e, b1r, w2, b2r, gm, bt,
                              c, BE)
    return acc
